# baseline (speedup 1.0000x reference)
import functools

import jax
import jax.numpy as jnp
from jax import lax
from jax.experimental import pallas as pl
from jax.experimental.pallas import tpu as pltpu
from jax.experimental.pallas import tpu_sc as plsc

NW = 32
C = 128
EPS = 1e-8


def _rsqrt(s):
    b = lax.bitcast_convert_type(s, jnp.int32)
    y = lax.bitcast_convert_type(jnp.int32(0x5F3759DF) - (b >> 1), jnp.float32)
    h = 0.5 * s
    for _ in range(3):
        y = y * (1.5 - h * y * y)
    return y


def _load_xyz(rows_ref, g):
    ridx = lax.iota(jnp.int32, 16) + g * 16
    x = plsc.load_gather(rows_ref, [ridx, jnp.full((16,), 0, jnp.int32)])
    y = plsc.load_gather(rows_ref, [ridx, jnp.full((16,), 1, jnp.int32)])
    z = plsc.load_gather(rows_ref, [ridx, jnp.full((16,), 2, jnp.int32)])
    return x, y, z


def _term_loop(nc, wid, coords, comb_hbm, S, P, comb_v, rows_v,
               sem_comb, sem_data, acc, compute_group):
    W = (S + P) * C

    def cslice(b, k):
        return comb_v[b].at[pl.ds(k * C, C)]

    def issue_comb(c, b):
        pltpu.async_copy(comb_hbm.at[wid, c], comb_v[b].at[pl.ds(0, W)],
                         sem_comb[b])

    def wait_comb(c, b):
        pltpu.make_async_copy(comb_hbm.at[wid, c], comb_v[b].at[pl.ds(0, W)],
                              sem_comb[b]).wait()

    def issue_data(c, b):
        for s in range(S):
            pltpu.async_copy(coords.at[cslice(b, s)], rows_v[s][b],
                             sem_data[b])

    def wait_data(c, b):
        for s in range(S):
            pltpu.make_async_copy(coords.at[cslice(b, s)], rows_v[s][b],
                                  sem_data[b]).wait()

    def compute(b, acc):
        for g in range(C // 16):
            pars = [comb_v[b][pl.ds((S + p) * C + g * 16, 16)]
                    for p in range(P)]
            acc = acc + compute_group(g, b, pars)
        return acc

    pltpu.sync_copy(comb_hbm.at[wid, 0], comb_v[0].at[pl.ds(0, W)])
    issue_comb(1, 1)
    issue_data(0, 0)

    def body(j, acc):
        c = 2 * j
        for (cc, b) in ((c, 0), (c + 1, 1)):
            wait_comb(cc + 1, 1 - b)
            issue_data(cc + 1, 1 - b)
            wait_data(cc, b)
            issue_comb(cc + 2, b)
            acc = compute(b, acc)
        return acc

    acc = lax.fori_loop(0, nc // 2 - 1, body, acc)

    wait_comb(nc - 1, 1)
    issue_data(nc - 1, 1)
    wait_data(nc - 2, 0)
    acc = compute(0, acc)
    wait_data(nc - 1, 1)
    acc = compute(1, acc)
    return acc


def _f32(v):
    return lax.bitcast_convert_type(v, jnp.float32)


def _uff_sc(nc_b, nc_a, nc_t, nc_n):
    mesh = plsc.VectorSubcoreMesh(core_axis_name="c", subcore_axis_name="s",
                                  num_cores=2, num_subcores=16)

    @functools.partial(
        pl.kernel,
        out_type=jax.ShapeDtypeStruct((NW, 16), jnp.float32),
        mesh=mesh,
        compiler_params=pltpu.CompilerParams(
            needs_layout_passes=False, use_tc_tiling_on_sc=False),
        scratch_types=[
            [pltpu.VMEM((7 * C,), jnp.int32) for _ in range(2)],
            [[pltpu.VMEM((C, 3), jnp.float32) for _ in range(2)]
             for _ in range(4)],
            pltpu.VMEM((16,), jnp.float32),
            [pltpu.SemaphoreType.DMA for _ in range(2)],
            [pltpu.SemaphoreType.DMA for _ in range(2)],
        ],
    )
    def k(coords, bond_c, angle_c, tors_c, nonb_c,
          out, comb_v, rows_v, acc_v, sem_comb, sem_data):
        wid = lax.axis_index("s") * 2 + lax.axis_index("c")
        acc = jnp.zeros((16,), jnp.float32)

        def bond_group(g, b, pars):
            r0, hk = _f32(pars[0]), _f32(pars[1])
            xa, ya, za = _load_xyz(rows_v[0][b], g)
            xb, yb, zb = _load_xyz(rows_v[1][b], g)
            dx, dy, dz = xa - xb, ya - yb, za - zb
            s = dx * dx + dy * dy + dz * dz + EPS
            r = s * _rsqrt(s)
            dr = r - r0
            return hk * dr * dr

        acc = _term_loop(nc_b, wid, coords, bond_c, 2, 2, comb_v, rows_v,
                         sem_comb, sem_data, acc, bond_group)

        def angle_group(g, b, pars):
            ak, c0, c1, c2 = (_f32(p) for p in pars)
            xi, yi, zi = _load_xyz(rows_v[0][b], g)
            xj, yj, zj = _load_xyz(rows_v[1][b], g)
            xk, yk, zk = _load_xyz(rows_v[2][b], g)
            v1x, v1y, v1z = xi - xj, yi - yj, zi - zj
            v2x, v2y, v2z = xk - xj, yk - yj, zk - zj
            q1 = v1x * v1x + v1y * v1y + v1z * v1z + EPS
            q2 = v2x * v2x + v2y * v2y + v2z * v2z + EPS
            dt = v1x * v2x + v1y * v2y + v1z * v2z
            cos = jnp.clip(dt * _rsqrt(q1 * q2), -0.9999, 0.9999)
            return ak * (c0 + c1 * cos + c2 * (2.0 * cos * cos - 1.0))

        acc = _term_loop(nc_a, wid, coords, angle_c, 3, 4, comb_v, rows_v,
                         sem_comb, sem_data, acc, angle_group)

        def torsion_group(g, b, pars):
            hk, ct = _f32(pars[0]), _f32(pars[1])
            order = pars[2]
            x0, y0, z0 = _load_xyz(rows_v[0][b], g)
            x1, y1, z1 = _load_xyz(rows_v[1][b], g)
            x2, y2, z2 = _load_xyz(rows_v[2][b], g)
            x3, y3, z3 = _load_xyz(rows_v[3][b], g)
            b1x, b1y, b1z = x1 - x0, y1 - y0, z1 - z0
            b2x, b2y, b2z = x2 - x1, y2 - y1, z2 - z1
            b3x, b3y, b3z = x3 - x2, y3 - y2, z3 - z2
            c1x = b1y * b2z - b1z * b2y
            c1y = b1z * b2x - b1x * b2z
            c1z = b1x * b2y - b1y * b2x
            c2x = b2y * b3z - b2z * b3y
            c2y = b2z * b3x - b2x * b3z
            c2z = b2x * b3y - b2y * b3x
            m1 = c1x * c1x + c1y * c1y + c1z * c1z + EPS
            m2 = c2x * c2x + c2y * c2y + c2z * c2z + EPS
            dt = c1x * c2x + c1y * c2y + c1z * c2z
            cos = jnp.clip(dt * _rsqrt(m1 * m2), -0.9999, 0.9999)
            cos2 = 2.0 * cos * cos - 1.0
            cos3 = cos * (4.0 * cos * cos - 3.0)
            cosn = jnp.where(order == 1, cos,
                             jnp.where(order == 2, cos2, cos3))
            return hk * (1.0 - ct * cosn)

        acc = _term_loop(nc_t, wid, coords, tors_c, 4, 3, comb_v, rows_v,
                         sem_comb, sem_data, acc, torsion_group)

        def vdw_group(g, b, pars):
            rm, dd = _f32(pars[0]), _f32(pars[1])
            xa, ya, za = _load_xyz(rows_v[0][b], g)
            xb, yb, zb = _load_xyz(rows_v[1][b], g)
            dx, dy, dz = xa - xb, ya - yb, za - zb
            r2 = jnp.maximum(dx * dx + dy * dy + dz * dz + EPS, 0.25)
            t = (rm * rm) / r2
            x6 = t * t * t
            return dd * x6 * (x6 - 2.0)

        acc = _term_loop(nc_n, wid, coords, nonb_c, 2, 2, comb_v, rows_v,
                         sem_comb, sem_data, acc, vdw_group)

        acc_v[...] = acc
        pltpu.sync_copy(acc_v, out.at[wid])

    return k


def _prep_comb(idx_cols, par_cols, total):
    cols = []
    for a in idx_cols:
        a = jnp.concatenate([a, jnp.zeros((total - a.shape[0],), a.dtype)]) \
            if total > a.shape[0] else a
        cols.append(a.reshape(NW, -1, 1, C))
    for a in par_cols:
        if a.dtype != jnp.int32:
            a = lax.bitcast_convert_type(a, jnp.int32)
        a = jnp.concatenate([a, jnp.zeros((total - a.shape[0],), jnp.int32)]) \
            if total > a.shape[0] else a
        cols.append(a.reshape(NW, -1, 1, C))
    comb = jnp.concatenate(cols, axis=2)
    return comb.reshape(NW, comb.shape[1], -1)


def kernel(coords, bond_rest_length, bond_half_force_constant,
           angle_force_constant, angle_c0, angle_c1, angle_c2,
           torsion_half_force_constant, torsion_cos_term, vdw_minimum,
           vdw_well_depth, bond_index, angle_index, torsion_index,
           torsion_order, nonbond_index):
    unit = NW * C * 2

    def up(t):
        return max(2, (t + unit - 1) // unit) * unit

    NB, NA = bond_index.shape[0], angle_index.shape[0]
    NT, NP = torsion_index.shape[0], nonbond_index.shape[0]
    NBp, NAp, NTp, NPp = up(NB), up(NA), up(NT), up(NP)

    bond_c = _prep_comb([bond_index[:, 0], bond_index[:, 1]],
                        [bond_rest_length, bond_half_force_constant], NBp)
    angle_c = _prep_comb([angle_index[:, s] for s in range(3)],
                         [angle_force_constant, angle_c0, angle_c1, angle_c2],
                         NAp)
    tors_c = _prep_comb([torsion_index[:, s] for s in range(4)],
                        [torsion_half_force_constant, torsion_cos_term,
                         torsion_order], NTp)
    nonb_c = _prep_comb([nonbond_index[:, 0], nonbond_index[:, 1]],
                        [vdw_minimum, vdw_well_depth], NPp)

    k = _uff_sc(NBp // (NW * C), NAp // (NW * C), NTp // (NW * C),
                NPp // (NW * C))
    partials = k(coords, bond_c, angle_c, tors_c, nonb_c)
    return jnp.sum(partials)
